# Initial kernel scaffold; baseline (speedup 1.0000x reference)
import functools

import jax
import jax.numpy as jnp
from jax import lax
from jax.experimental import pallas as pl
from jax.experimental.pallas import tpu as pltpu
from jax.experimental.pallas import tpu_sc as plsc

VOCAB = 1000000
EMBED_DIM = 32

NC = 2
NS = 16
NW = NC * NS

B_TOTAL = 16384 * 50
B_PER_W = B_TOTAL // NW
CHUNK = 1600
N_CHUNKS = B_PER_W // CHUNK


def _gather_body(idx_hbm, table_hbm, out_hbm, idx_v, rows_v, sem):
    wid = lax.axis_index("s") * NC + lax.axis_index("c")
    base = wid * B_PER_W

    def chunk_step(j, carry):
        off = base + j * CHUNK
        pltpu.sync_copy(idx_hbm.at[pl.ds(off, CHUNK)], idx_v)
        pltpu.async_copy(table_hbm.at[idx_v], rows_v, sem).wait()
        pltpu.sync_copy(rows_v, out_hbm.at[pl.ds(off, CHUNK)])
        return carry

    lax.fori_loop(0, N_CHUNKS, chunk_step, 0)


@jax.jit
def _sc_gather(flat_idx, weight):
    mesh = plsc.VectorSubcoreMesh(core_axis_name="c", subcore_axis_name="s")
    return pl.kernel(
        _gather_body,
        out_type=jax.ShapeDtypeStruct((B_TOTAL, EMBED_DIM), jnp.float32),
        mesh=mesh,
        scratch_types=[
            pltpu.VMEM((CHUNK,), jnp.int32),
            pltpu.VMEM((CHUNK, EMBED_DIM), jnp.float32),
            pltpu.SemaphoreType.DMA,
        ],
    )(flat_idx, weight)


def kernel(input_ids, weight):
    n, s = input_ids.shape
    flat_idx = input_ids.reshape(-1).astype(jnp.int32)
    out = _sc_gather(flat_idx, weight)
    return out.reshape(n, s, EMBED_DIM)


# speedup vs baseline: 1.1030x; 1.1030x over previous
import functools

import jax
import jax.numpy as jnp
from jax import lax
from jax.experimental import pallas as pl
from jax.experimental.pallas import tpu as pltpu
from jax.experimental.pallas import tpu_sc as plsc

VOCAB = 1000000
EMBED_DIM = 32

NC = 2
NS = 16
NW = NC * NS

B_TOTAL = 16384 * 50
B_PER_W = B_TOTAL // NW
CHUNK = 1600
N_CHUNKS = B_PER_W // CHUNK


def _gather_body(idx_hbm, table_hbm, out_hbm, idx_v, rows_v, sem):
    wid = lax.axis_index("s") * NC + lax.axis_index("c")
    base = wid * B_PER_W

    def chunk_step(j, carry):
        off = base + j * CHUNK
        pltpu.sync_copy(idx_hbm.at[pl.ds(off, CHUNK)], idx_v)
        pltpu.async_copy(table_hbm.at[idx_v], rows_v, sem).wait()
        pltpu.sync_copy(rows_v, out_hbm.at[pl.ds(off, CHUNK)])
        return carry

    lax.fori_loop(0, N_CHUNKS, chunk_step, 0)


@jax.jit
def _sc_gather(flat_idx, weight):
    mesh = plsc.VectorSubcoreMesh(core_axis_name="c", subcore_axis_name="s")
    return pl.kernel(
        _gather_body,
        out_type=jax.ShapeDtypeStruct((B_TOTAL, EMBED_DIM), jnp.float32),
        mesh=mesh,
        scratch_types=[
            pltpu.VMEM((CHUNK,), jnp.int32),
            pltpu.VMEM((CHUNK, EMBED_DIM), jnp.float32),
            pltpu.SemaphoreType.DMA,
        ],
        compiler_params=pltpu.CompilerParams(use_tc_tiling_on_sc=False),
    )(flat_idx, weight)


def kernel(input_ids, weight):
    n, s = input_ids.shape
    flat_idx = input_ids.reshape(-1).astype(jnp.int32)
    out = _sc_gather(flat_idx, weight)
    return out.reshape(n, s, EMBED_DIM)


# speedup vs baseline: 1.1135x; 1.0094x over previous
import functools

import jax
import jax.numpy as jnp
from jax import lax
from jax.experimental import pallas as pl
from jax.experimental.pallas import tpu as pltpu
from jax.experimental.pallas import tpu_sc as plsc

VOCAB = 1000000
EMBED_DIM = 32

NC = 2
NS = 16
NW = NC * NS

B_TOTAL = 16384 * 50
B_PER_W = B_TOTAL // NW
CHUNK = 1600
N_CHUNKS = B_PER_W // CHUNK


def _gather_body(idx_hbm, table_hbm, out_hbm,
                 idx_v0, idx_v1, rows_v0, rows_v1,
                 sem_g0, sem_g1, sem_w0, sem_w1):
    wid = lax.axis_index("s") * NC + lax.axis_index("c")
    base = wid * B_PER_W

    idx_v = [idx_v0, idx_v1]
    rows_v = [rows_v0, rows_v1]
    sem_g = [sem_g0, sem_g1]
    sem_w = [sem_w0, sem_w1]

    pltpu.sync_copy(idx_hbm.at[pl.ds(base, CHUNK)], idx_v[0])
    gathers = [None] * N_CHUNKS
    writes = [None] * N_CHUNKS
    gathers[0] = pltpu.async_copy(table_hbm.at[idx_v[0]], rows_v[0], sem_g[0])
    for j in range(N_CHUNKS):
        b = j % 2
        nb = 1 - b
        if j + 1 < N_CHUNKS:
            off_n = base + (j + 1) * CHUNK
            pltpu.sync_copy(idx_hbm.at[pl.ds(off_n, CHUNK)], idx_v[nb])
            if j >= 1:
                writes[j - 1].wait()
            gathers[j + 1] = pltpu.async_copy(
                table_hbm.at[idx_v[nb]], rows_v[nb], sem_g[nb])
        gathers[j].wait()
        off = base + j * CHUNK
        writes[j] = pltpu.async_copy(
            rows_v[b], out_hbm.at[pl.ds(off, CHUNK)], sem_w[b])
    writes[N_CHUNKS - 2].wait()
    writes[N_CHUNKS - 1].wait()


@jax.jit
def _sc_gather(flat_idx, weight):
    mesh = plsc.VectorSubcoreMesh(core_axis_name="c", subcore_axis_name="s")
    return pl.kernel(
        _gather_body,
        out_type=jax.ShapeDtypeStruct((B_TOTAL, EMBED_DIM), jnp.float32),
        mesh=mesh,
        scratch_types=[
            pltpu.VMEM((CHUNK,), jnp.int32),
            pltpu.VMEM((CHUNK,), jnp.int32),
            pltpu.VMEM((CHUNK, EMBED_DIM), jnp.float32),
            pltpu.VMEM((CHUNK, EMBED_DIM), jnp.float32),
            pltpu.SemaphoreType.DMA,
            pltpu.SemaphoreType.DMA,
            pltpu.SemaphoreType.DMA,
            pltpu.SemaphoreType.DMA,
        ],
        compiler_params=pltpu.CompilerParams(use_tc_tiling_on_sc=False),
    )(flat_idx, weight)


def kernel(input_ids, weight):
    n, s = input_ids.shape
    flat_idx = input_ids.reshape(-1).astype(jnp.int32)
    out = _sc_gather(flat_idx, weight)
    return out.reshape(n, s, EMBED_DIM)
